# baseline (speedup 1.0000x reference)
import functools

import jax
import jax.numpy as jnp
import numpy as np
from jax import lax
from jax.experimental import pallas as pl
from jax.experimental.pallas import tpu as pltpu
from jax.experimental.pallas import tpu_sc as plsc

N = 10000
E = 320000
D = 128
NPAD = 10240
BLK = 1024
GRID = NPAD // BLK
NW = 32
CH = 128
CPT = 78
EMAIN = NW * CPT * CH
XTILES = (E - EMAIN) // CH
IPT = CPT * CH
IBUF = IPT + CH
ROWS_PT = NPAD // 16
INV_SQRT_D = 1.0 / np.sqrt(float(D))
HALF_INV_SQRT_D = 0.5 / np.sqrt(float(D))


def _matmuls_tail(h, wq, bq, wk, bk, wv, bv, ws, bs,
                  q_o, k_o, v_o, xs_o, qn2_o, k2_o):
    q = jnp.dot(h, wq[...], preferred_element_type=jnp.float32) + bq[...]
    k = jnp.dot(h, wk[...], preferred_element_type=jnp.float32) + bk[...]
    v = jnp.dot(h, wv[...], preferred_element_type=jnp.float32) + bv[...]
    xs = jnp.dot(h, ws[...], preferred_element_type=jnp.float32) + bs[...]
    q_o[...] = q
    k_o[...] = k
    v_o[...] = v
    xs_o[...] = xs
    qn2_o[...] = jnp.sum(q * q, axis=1, keepdims=True)
    kmax = jnp.max(jnp.sum(k * k, axis=1)).reshape(1, 1)
    i = pl.program_id(0)

    @pl.when(i == 0)
    def _():
        k2_o[...] = kmax

    @pl.when(i != 0)
    def _():
        k2_o[...] = jnp.maximum(k2_o[...], kmax)


def _prep1_body(x, wq, bq, wk, bk, wv, bv, ws, bs,
                q_o, k_o, v_o, xs_o, qn2_o, k2_o):
    _matmuls_tail(x[...], wq, bq, wk, bk, wv, bv, ws, bs,
                  q_o, k_o, v_o, xs_o, qn2_o, k2_o)


def _prep23_body(n0, n1, s0, s1, xsp, bsp,
                 wq, bq, wk, bk, wv, bv, ws, bs,
                 q_o, k_o, v_o, xs_o, qn2_o, k2_o):
    agg = (n0[...] + n1[...]) / (s0[...] + s1[...] + 1e-16)
    h = jnp.maximum(agg + xsp[...] + bsp[...], 0.0)
    _matmuls_tail(h, wq, bq, wk, bk, wv, bv, ws, bs,
                  q_o, k_o, v_o, xs_o, qn2_o, k2_o)


def _final_body(n0, n1, s0, s1, xsp, bsp, out):
    agg = (n0[...] + n1[...]) / (s0[...] + s1[...] + 1e-16)
    z = agg + xsp[...] + bsp[...]
    m = jnp.max(z, axis=1, keepdims=True)
    lse = m + jnp.log(jnp.sum(jnp.exp(z - m), axis=1, keepdims=True))
    out[...] = z - lse


_ROWS = pl.BlockSpec((BLK, D), lambda i: (i, 0))
_COL1 = pl.BlockSpec((BLK, 1), lambda i: (i, 0))
_WSPEC = pl.BlockSpec((D, D), lambda i: (0, 0))
_BSPEC = pl.BlockSpec((1, D), lambda i: (0, 0))
_SCALAR = pl.BlockSpec((1, 1), lambda i: (0, 0))

_PREP_OUT_SHAPES = (
    [jax.ShapeDtypeStruct((NPAD, D), jnp.float32)] * 4
    + [jax.ShapeDtypeStruct((NPAD, 1), jnp.float32),
       jax.ShapeDtypeStruct((1, 1), jnp.float32)]
)
_PREP_OUT_SPECS = [_ROWS] * 4 + [_COL1, _SCALAR]

_prep1 = pl.pallas_call(
    _prep1_body,
    grid=(GRID,),
    in_specs=[_ROWS] + [_WSPEC, _BSPEC] * 4,
    out_specs=_PREP_OUT_SPECS,
    out_shape=_PREP_OUT_SHAPES,
)

_prep23 = pl.pallas_call(
    _prep23_body,
    grid=(GRID,),
    in_specs=[_ROWS, _ROWS, _COL1, _COL1, _ROWS, _BSPEC] + [_WSPEC, _BSPEC] * 4,
    out_specs=_PREP_OUT_SPECS,
    out_shape=_PREP_OUT_SHAPES,
)

_final = pl.pallas_call(
    _final_body,
    grid=(GRID,),
    in_specs=[_ROWS, _ROWS, _COL1, _COL1, _ROWS, _BSPEC],
    out_specs=_ROWS,
    out_shape=jax.ShapeDtypeStruct((NPAD, D), jnp.float32),
)


_MESH = plsc.VectorSubcoreMesh(core_axis_name="c", subcore_axis_name="s")


@functools.partial(
    pl.kernel,
    mesh=_MESH,
    out_type=jax.ShapeDtypeStruct((E,), jnp.float32),
    scratch_types=[
        pltpu.VMEM((IBUF,), jnp.int32),
        pltpu.VMEM((IBUF,), jnp.int32),
        pltpu.VMEM((CH, D), jnp.float32),
        pltpu.VMEM((CH, D), jnp.float32),
        pltpu.VMEM((CH, D), jnp.float32),
        pltpu.VMEM((CH, D), jnp.float32),
        pltpu.VMEM((CH, D), jnp.float32),
        pltpu.VMEM((CH, D), jnp.float32),
        pltpu.VMEM((CH,), jnp.float32),
        pltpu.VMEM((CH,), jnp.float32),
        pltpu.VMEM((CH,), jnp.float32),
        pltpu.VMEM((CH,), jnp.float32),
        pltpu.VMEM((CH,), jnp.float32),
        pltpu.VMEM((CH,), jnp.float32),
        pltpu.VMEM((512,), jnp.float32),
        pltpu.VMEM((16,), jnp.float32),
        pltpu.SemaphoreType.DMA,
        pltpu.SemaphoreType.DMA,
        pltpu.SemaphoreType.DMA,
        pltpu.SemaphoreType.DMA,
        pltpu.SemaphoreType.DMA,
        pltpu.SemaphoreType.DMA,
    ],
)
def _sc_score(q_hbm, k_hbm, qn2_hbm, k2_hbm, src_hbm, dst_hbm,
              e_out,
              srcall, dstall, qr0, qr1, qr2, kr0, kr1, kr2, q20, q21, q22,
              eb0, eb1, eb2, fb, k2b,
              sem_g0, sem_g1, sem_g2, sem_w0, sem_w1, sem_w2):
    cid = lax.axis_index("c")
    sid = lax.axis_index("s")
    wid = sid * 2 + cid
    base = wid * IPT

    pltpu.sync_copy(src_hbm.at[pl.ds(base, IPT)], srcall.at[pl.ds(0, IPT)])
    pltpu.sync_copy(dst_hbm.at[pl.ds(base, IPT)], dstall.at[pl.ds(0, IPT)])
    pltpu.sync_copy(k2_hbm, k2b)
    k2v = k2b[...]

    @pl.when(wid < XTILES)
    def _():
        xoff = EMAIN + wid * CH
        pltpu.sync_copy(src_hbm.at[pl.ds(xoff, CH)], srcall.at[pl.ds(IPT, CH)])
        pltpu.sync_copy(dst_hbm.at[pl.ds(xoff, CH)], dstall.at[pl.ds(IPT, CH)])

    zv = jnp.zeros((16,), jnp.float32)
    for z in range(32):
        fb[pl.ds(z * 16, 16)] = zv

    bufs = ((qr0, kr0, q20, eb0, sem_g0, sem_w0),
            (qr1, kr1, q21, eb1, sem_g1, sem_w1),
            (qr2, kr2, q22, eb2, sem_g2, sem_w2))
    lane = lax.iota(jnp.int32, 16)

    def gathers(lj, b):
        qr, kr, q2, _, sem_g, _ = bufs[b]
        ssl = srcall.at[pl.ds(lj * CH, CH)]
        dsl = dstall.at[pl.ds(lj * CH, CH)]
        return (pltpu.make_async_copy(q_hbm.at[dsl], qr, sem_g),
                pltpu.make_async_copy(k_hbm.at[ssl], kr, sem_g),
                pltpu.make_async_copy(qn2_hbm.at[dsl], q2, sem_g))

    def ewrite(off, b):
        _, _, _, eb, _, sem_w = bufs[b]
        return (pltpu.make_async_copy(eb, e_out.at[pl.ds(off, CH)], sem_w),)

    def start(cps):
        for cp in cps:
            cp.start()

    def wait(cps):
        for cp in cps:
            cp.wait()

    def compute(b):
        qr, kr, q2, eb, _, _ = bufs[b]

        def group(g, _):
            gb = g * 16
            mbv = (q2[pl.ds(gb, 16)] + k2v) * HALF_INV_SQRT_D
            accs = []
            for l in range(16):
                i = gb + l
                a0 = qr[i, pl.ds(0, 16)] * kr[i, pl.ds(0, 16)]
                a1 = qr[i, pl.ds(16, 16)] * kr[i, pl.ds(16, 16)]
                for c8 in range(2, D // 16, 2):
                    a0 = a0 + qr[i, pl.ds(c8 * 16, 16)] * kr[i, pl.ds(c8 * 16, 16)]
                    a1 = a1 + qr[i, pl.ds(c8 * 16 + 16, 16)] * kr[i, pl.ds(c8 * 16 + 16, 16)]
                accs.append(a0 + a1)
            for sh in (8, 4, 2, 1):
                for l in range(16):
                    fb[pl.ds(l * 32, 16)] = accs[l]
                for l in range(16):
                    accs[l] = accs[l] + fb[pl.ds(l * 32 + sh, 16)]
            ex = jnp.zeros((16,), jnp.float32)
            for l in range(16):
                ex = ex + jnp.where(lane == l, accs[l][0], 0.0)
            eb[pl.ds(gb, 16)] = jnp.exp(ex * INV_SQRT_D - mbv)
            return 0

        lax.fori_loop(0, CH // 16, group, 0)

    def retire(j, b):
        wait(gathers(j, b))
        compute(b)
        start(ewrite(base + j * CH, b))

    def body(jj, _):
        for b in (0, 1, 2):
            j = 3 * jj + b

            @pl.when(jj > 0)
            def _():
                wait(ewrite(base + (j - 3) * CH, b))

            start(gathers(j, b))

            if b > 0:
                retire(j - 1, b - 1)
            else:
                pl.when(jj > 0)(lambda: retire(j - 1, 2))
        return 0

    lax.fori_loop(0, CPT // 3, body, 0)

    last = CPT - 1
    retire(last, last % 3)
    wait(ewrite(base + (last - 2) * CH, (last - 2) % 3))
    wait(ewrite(base + (last - 1) * CH, (last - 1) % 3))
    wait(ewrite(base + last * CH, last % 3))

    @pl.when(wid < XTILES)
    def _():
        xoff = EMAIN + wid * CH
        g = gathers(CPT, 0)
        start(g)
        wait(g)
        compute(0)
        w = ewrite(xoff, 0)
        start(w)
        wait(w)


@functools.partial(
    pl.kernel,
    mesh=_MESH,
    out_type=[
        jax.ShapeDtypeStruct((2, NPAD, D), jnp.float32),
        jax.ShapeDtypeStruct((2, NPAD), jnp.float32),
    ],
    scratch_types=[
        pltpu.VMEM((IBUF,), jnp.int32),
        pltpu.VMEM((CH, D), jnp.float32),
        pltpu.VMEM((CH, D), jnp.float32),
        pltpu.VMEM((CH,), jnp.float32),
        pltpu.VMEM((CH,), jnp.float32),
        pltpu.VMEM((CH,), jnp.int32),
        pltpu.VMEM((CH,), jnp.int32),
        pltpu.VMEM_SHARED((NPAD, D), jnp.float32),
        pltpu.VMEM_SHARED((NPAD,), jnp.float32),
        pltpu.SemaphoreType.DMA,
        pltpu.SemaphoreType.DMA,
        pltpu.SemaphoreType.DMA,
        pltpu.SemaphoreType.DMA,
    ],
)
def _sc_scatter(v_hbm, e_hbm, src_hbm, dst_hbm,
                nacc_out, s_out,
                srcall, vb0, vb1, eb0, eb1, db0, db1, nacc_sh, s_sh,
                sem_g0, sem_g1, sem_s0, sem_s1):
    cid = lax.axis_index("c")
    sid = lax.axis_index("s")
    wid = sid * 2 + cid
    base = wid * IPT
    r0 = sid * ROWS_PT

    pltpu.sync_copy(src_hbm.at[pl.ds(base, IPT)], srcall.at[pl.ds(0, IPT)])

    @pl.when(wid < XTILES)
    def _():
        xoff = EMAIN + wid * CH
        pltpu.sync_copy(src_hbm.at[pl.ds(xoff, CH)], srcall.at[pl.ds(IPT, CH)])

    bufs = ((vb0, eb0, db0, sem_g0, sem_s0),
            (vb1, eb1, db1, sem_g1, sem_s1))

    zv = jnp.zeros((16,), jnp.float32)

    def _zero_row(i, _):
        for c8 in range(D // 16):
            vb0[i, pl.ds(c8 * 16, 16)] = zv
        return 0

    lax.fori_loop(0, CH, _zero_row, 0)
    for g in range(CH // 16):
        eb0[pl.ds(g * 16, 16)] = zv
    for m in range(ROWS_PT // CH):
        pltpu.sync_copy(vb0, nacc_sh.at[pl.ds(r0 + m * CH, CH)])
        pltpu.sync_copy(eb0, s_sh.at[pl.ds(r0 + m * CH, CH)])
    plsc.subcore_barrier()

    def gathers(lj, off, b):
        vb, eb, db, sem_g, _ = bufs[b]
        ssl = srcall.at[pl.ds(lj * CH, CH)]
        return (pltpu.make_async_copy(v_hbm.at[ssl], vb, sem_g),
                pltpu.make_async_copy(e_hbm.at[pl.ds(off, CH)], eb, sem_g),
                pltpu.make_async_copy(dst_hbm.at[pl.ds(off, CH)], db, sem_g))

    def scatters(b):
        vb, eb, db, _, sem_s = bufs[b]
        return (pltpu.make_async_copy(vb, nacc_sh.at[db], sem_s),
                pltpu.make_async_copy(eb, s_sh.at[db], sem_s))

    def start(cps):
        for cp in cps:
            cp.start()

    def wait(cps):
        for cp in cps:
            cp.wait()

    def scale(b):
        vb, eb, _, _, _ = bufs[b]

        def sbody(g, _):
            ev = eb[pl.ds(g * 16, 16)]
            for l in range(16):
                a = ev[l]
                i = g * 16 + l
                for c8 in range(D // 16):
                    sl = pl.ds(c8 * 16, 16)
                    vb[i, sl] = vb[i, sl] * a
            return 0

        lax.fori_loop(0, CH // 16, sbody, 0)

    def retire(j, b):
        wait(gathers(j, base + j * CH, b))
        scale(b)
        vb, eb, db, _, sem_s = bufs[b]
        pltpu.async_copy(vb, nacc_sh.at[db], sem_s, add=True)
        pltpu.async_copy(eb, s_sh.at[db], sem_s, add=True)

    def body(jj, _):
        for b in (0, 1):
            j = 2 * jj + b
            first = (jj == 0) if b == 0 else None

            @pl.when(jj > 0)
            def _():
                wait(scatters(b))

            start(gathers(j, base + j * CH, b))

            if first is None:
                retire(j - 1, 1 - b)
            else:
                pl.when(jj > 0)(lambda: retire(j - 1, 1 - b))
        return 0

    lax.fori_loop(0, CPT // 2, body, 0)

    last = CPT - 1
    retire(last, last % 2)
    wait(scatters((last - 1) % 2))
    wait(scatters(last % 2))

    @pl.when(wid < XTILES)
    def _():
        xoff = EMAIN + wid * CH
        g = gathers(CPT, xoff, 0)
        start(g)
        wait(g)
        scale(0)
        vb, eb, db, _, _ = bufs[0]
        pltpu.sync_copy(vb, nacc_sh.at[db], add=True)
        pltpu.sync_copy(eb, s_sh.at[db], add=True)

    plsc.subcore_barrier()
    pltpu.sync_copy(nacc_sh.at[pl.ds(r0, ROWS_PT)],
                    nacc_out.at[cid, pl.ds(r0, ROWS_PT)])
    pltpu.sync_copy(s_sh.at[pl.ds(r0, ROWS_PT)],
                    s_out.at[cid, pl.ds(r0, ROWS_PT)])


def _layer_sc(prep_outs, src, dst):
    q, k, v, xs, qn2, k2 = prep_outs
    qn2f = qn2.reshape(NPAD)
    k2v = jnp.broadcast_to(k2.reshape(()), (16,))
    e = _sc_score(q, k, qn2f, k2v, src, dst)
    nacc, sacc = _sc_scatter(v, e, src, dst)
    s_col = sacc.reshape(2, NPAD, 1)
    return nacc[0], nacc[1], s_col[0], s_col[1], xs


def kernel(x, edge_index,
           Wq1, bq1, Wk1, bk1, Wv1, bv1, Ws1, bs1,
           Wq2, bq2, Wk2, bk2, Wv2, bv2, Ws2, bs2,
           Wq3, bq3, Wk3, bk3, Wv3, bv3, Ws3, bs3):
    src = edge_index[0]
    dst = edge_index[1]
    xp = jnp.pad(x, ((0, NPAD - N), (0, 0)))

    def r1(b):
        return b.reshape(1, D)

    p1 = _prep1(xp, Wq1, r1(bq1), Wk1, r1(bk1), Wv1, r1(bv1), Ws1, r1(bs1))
    n0, n1, s0, s1, xs1 = _layer_sc(p1, src, dst)

    p2 = _prep23(n0, n1, s0, s1, xs1, r1(bs1),
                 Wq2, r1(bq2), Wk2, r1(bk2), Wv2, r1(bv2), Ws2, r1(bs2))
    n0, n1, s0, s1, xs2 = _layer_sc(p2, src, dst)

    p3 = _prep23(n0, n1, s0, s1, xs2, r1(bs2),
                 Wq3, r1(bq3), Wk3, r1(bk3), Wv3, r1(bv3), Ws3, r1(bs3))
    n0, n1, s0, s1, xs3 = _layer_sc(p3, src, dst)

    out = _final(n0, n1, s0, s1, xs3, r1(bs3))
    return out[:N]


# speedup vs baseline: 1.0125x; 1.0125x over previous
import functools

import jax
import jax.numpy as jnp
import numpy as np
from jax import lax
from jax.experimental import pallas as pl
from jax.experimental.pallas import tpu as pltpu
from jax.experimental.pallas import tpu_sc as plsc

N = 10000
E = 320000
D = 128
NPAD = 10240
BLK = 1024
GRID = NPAD // BLK
NW = 32
CH = 128
CPT = 78
EMAIN = NW * CPT * CH
XTILES = (E - EMAIN) // CH
IPT = CPT * CH
IBUF = IPT + CH
ROWS_PT = NPAD // 16
INV_SQRT_D = 1.0 / np.sqrt(float(D))
HALF_INV_SQRT_D = 0.5 / np.sqrt(float(D))


def _matmuls_tail(h, wq, bq, wk, bk, wv, bv, ws, bs,
                  q_o, k_o, v_o, xs_o, qn2_o, k2_o):
    q = jnp.dot(h, wq[...], preferred_element_type=jnp.float32) + bq[...]
    k = jnp.dot(h, wk[...], preferred_element_type=jnp.float32) + bk[...]
    v = jnp.dot(h, wv[...], preferred_element_type=jnp.float32) + bv[...]
    xs = jnp.dot(h, ws[...], preferred_element_type=jnp.float32) + bs[...]
    q_o[...] = q
    k_o[...] = k
    v_o[...] = v
    xs_o[...] = xs
    qn2_o[...] = jnp.sum(q * q, axis=1, keepdims=True)
    kmax = jnp.max(jnp.sum(k * k, axis=1)).reshape(1, 1)
    i = pl.program_id(0)

    @pl.when(i == 0)
    def _():
        k2_o[...] = kmax

    @pl.when(i != 0)
    def _():
        k2_o[...] = jnp.maximum(k2_o[...], kmax)


def _prep1_body(x, wq, bq, wk, bk, wv, bv, ws, bs,
                q_o, k_o, v_o, xs_o, qn2_o, k2_o):
    _matmuls_tail(x[...], wq, bq, wk, bk, wv, bv, ws, bs,
                  q_o, k_o, v_o, xs_o, qn2_o, k2_o)


def _prep23_body(n, s0, s1, xsp, bsp,
                 wq, bq, wk, bk, wv, bv, ws, bs,
                 q_o, k_o, v_o, xs_o, qn2_o, k2_o):
    agg = (n[0] + n[1]) / (s0[...] + s1[...] + 1e-16)
    h = jnp.maximum(agg + xsp[...] + bsp[...], 0.0)
    _matmuls_tail(h, wq, bq, wk, bk, wv, bv, ws, bs,
                  q_o, k_o, v_o, xs_o, qn2_o, k2_o)


def _final_body(n, s0, s1, xsp, bsp, out):
    agg = (n[0] + n[1]) / (s0[...] + s1[...] + 1e-16)
    z = agg + xsp[...] + bsp[...]
    m = jnp.max(z, axis=1, keepdims=True)
    lse = m + jnp.log(jnp.sum(jnp.exp(z - m), axis=1, keepdims=True))
    out[...] = z - lse


_ROWS = pl.BlockSpec((BLK, D), lambda i: (i, 0))
_COL1 = pl.BlockSpec((BLK, 1), lambda i: (i, 0))
_WSPEC = pl.BlockSpec((D, D), lambda i: (0, 0))
_BSPEC = pl.BlockSpec((1, D), lambda i: (0, 0))
_SCALAR = pl.BlockSpec((1, 1), lambda i: (0, 0))
_NROWS = pl.BlockSpec((2, BLK, D), lambda i: (0, i, 0))

_PREP_OUT_SHAPES = (
    [jax.ShapeDtypeStruct((NPAD, D), jnp.float32)] * 4
    + [jax.ShapeDtypeStruct((NPAD, 1), jnp.float32),
       jax.ShapeDtypeStruct((1, 1), jnp.float32)]
)
_PREP_OUT_SPECS = [_ROWS] * 4 + [_COL1, _SCALAR]

_prep1 = pl.pallas_call(
    _prep1_body,
    grid=(GRID,),
    in_specs=[_ROWS] + [_WSPEC, _BSPEC] * 4,
    out_specs=_PREP_OUT_SPECS,
    out_shape=_PREP_OUT_SHAPES,
)

_prep23 = pl.pallas_call(
    _prep23_body,
    grid=(GRID,),
    in_specs=[_NROWS, _COL1, _COL1, _ROWS, _BSPEC] + [_WSPEC, _BSPEC] * 4,
    out_specs=_PREP_OUT_SPECS,
    out_shape=_PREP_OUT_SHAPES,
)

_final = pl.pallas_call(
    _final_body,
    grid=(GRID,),
    in_specs=[_NROWS, _COL1, _COL1, _ROWS, _BSPEC],
    out_specs=_ROWS,
    out_shape=jax.ShapeDtypeStruct((NPAD, D), jnp.float32),
)


_MESH = plsc.VectorSubcoreMesh(core_axis_name="c", subcore_axis_name="s")


@functools.partial(
    pl.kernel,
    mesh=_MESH,
    out_type=jax.ShapeDtypeStruct((E,), jnp.float32),
    scratch_types=[
        pltpu.VMEM((IBUF,), jnp.int32),
        pltpu.VMEM((IBUF,), jnp.int32),
        pltpu.VMEM((CH, D), jnp.float32),
        pltpu.VMEM((CH, D), jnp.float32),
        pltpu.VMEM((CH, D), jnp.float32),
        pltpu.VMEM((CH, D), jnp.float32),
        pltpu.VMEM((CH,), jnp.float32),
        pltpu.VMEM((CH,), jnp.float32),
        pltpu.VMEM((CH,), jnp.float32),
        pltpu.VMEM((CH,), jnp.float32),
        pltpu.VMEM((512,), jnp.float32),
        pltpu.VMEM((16,), jnp.float32),
        pltpu.SemaphoreType.DMA,
        pltpu.SemaphoreType.DMA,
        pltpu.SemaphoreType.DMA,
        pltpu.SemaphoreType.DMA,
    ],
)
def _sc_score(q_hbm, k_hbm, qn2_hbm, k2_hbm, src_hbm, dst_hbm,
              e_out,
              srcall, dstall, qr0, qr1, kr0, kr1, q20, q21,
              eb0, eb1, fb, k2b,
              sem_g0, sem_g1, sem_w0, sem_w1):
    cid = lax.axis_index("c")
    sid = lax.axis_index("s")
    wid = sid * 2 + cid
    base = wid * IPT

    pltpu.sync_copy(src_hbm.at[pl.ds(base, IPT)], srcall.at[pl.ds(0, IPT)])
    pltpu.sync_copy(dst_hbm.at[pl.ds(base, IPT)], dstall.at[pl.ds(0, IPT)])
    pltpu.sync_copy(k2_hbm, k2b)
    k2v = k2b[...]

    @pl.when(wid < XTILES)
    def _():
        xoff = EMAIN + wid * CH
        pltpu.sync_copy(src_hbm.at[pl.ds(xoff, CH)], srcall.at[pl.ds(IPT, CH)])
        pltpu.sync_copy(dst_hbm.at[pl.ds(xoff, CH)], dstall.at[pl.ds(IPT, CH)])

    zv = jnp.zeros((16,), jnp.float32)
    for z in range(32):
        fb[pl.ds(z * 16, 16)] = zv

    bufs = ((qr0, kr0, q20, eb0, sem_g0, sem_w0),
            (qr1, kr1, q21, eb1, sem_g1, sem_w1))
    lane = lax.iota(jnp.int32, 16)

    def gathers(lj, b):
        qr, kr, q2, _, sem_g, _ = bufs[b]
        ssl = srcall.at[pl.ds(lj * CH, CH)]
        dsl = dstall.at[pl.ds(lj * CH, CH)]
        return (pltpu.make_async_copy(q_hbm.at[dsl], qr, sem_g),
                pltpu.make_async_copy(k_hbm.at[ssl], kr, sem_g),
                pltpu.make_async_copy(qn2_hbm.at[dsl], q2, sem_g))

    def ewrite(off, b):
        _, _, _, eb, _, sem_w = bufs[b]
        return (pltpu.make_async_copy(eb, e_out.at[pl.ds(off, CH)], sem_w),)

    def start(cps):
        for cp in cps:
            cp.start()

    def wait(cps):
        for cp in cps:
            cp.wait()

    def compute(b):
        qr, kr, q2, eb, _, _ = bufs[b]

        def group(g, _):
            gb = g * 16
            mbv = (q2[pl.ds(gb, 16)] + k2v) * HALF_INV_SQRT_D
            accs = []
            for l in range(16):
                i = gb + l
                a0 = qr[i, pl.ds(0, 16)] * kr[i, pl.ds(0, 16)]
                a1 = qr[i, pl.ds(16, 16)] * kr[i, pl.ds(16, 16)]
                for c8 in range(2, D // 16, 2):
                    a0 = a0 + qr[i, pl.ds(c8 * 16, 16)] * kr[i, pl.ds(c8 * 16, 16)]
                    a1 = a1 + qr[i, pl.ds(c8 * 16 + 16, 16)] * kr[i, pl.ds(c8 * 16 + 16, 16)]
                accs.append(a0 + a1)
            for sh in (8, 4, 2, 1):
                for l in range(16):
                    fb[pl.ds(l * 32, 16)] = accs[l]
                for l in range(16):
                    accs[l] = accs[l] + fb[pl.ds(l * 32 + sh, 16)]
            ex = jnp.zeros((16,), jnp.float32)
            for l in range(16):
                ex = ex + jnp.where(lane == l, accs[l][0], 0.0)
            eb[pl.ds(gb, 16)] = jnp.exp(ex * INV_SQRT_D - mbv)
            return 0

        lax.fori_loop(0, CH // 16, group, 0)

    def retire(j, b):
        wait(gathers(j, b))
        compute(b)
        start(ewrite(base + j * CH, b))

    def body(jj, _):
        for b in (0, 1):
            j = 2 * jj + b
            first = (jj == 0) if b == 0 else None

            @pl.when(jj > 0)
            def _():
                wait(ewrite(base + (j - 2) * CH, b))

            start(gathers(j, b))

            if first is None:
                retire(j - 1, 1 - b)
            else:
                pl.when(jj > 0)(lambda: retire(j - 1, 1 - b))
        return 0

    lax.fori_loop(0, CPT // 2, body, 0)

    last = CPT - 1
    retire(last, last % 2)
    wait(ewrite(base + (last - 1) * CH, (last - 1) % 2))
    wait(ewrite(base + last * CH, last % 2))

    @pl.when(wid < XTILES)
    def _():
        xoff = EMAIN + wid * CH
        g = gathers(CPT, 0)
        start(g)
        wait(g)
        compute(0)
        w = ewrite(xoff, 0)
        start(w)
        wait(w)


@functools.partial(
    pl.kernel,
    mesh=_MESH,
    out_type=[
        jax.ShapeDtypeStruct((2, NPAD, D), jnp.float32),
        jax.ShapeDtypeStruct((2, NPAD), jnp.float32),
    ],
    scratch_types=[
        pltpu.VMEM((IBUF,), jnp.int32),
        pltpu.VMEM((CH, D), jnp.float32),
        pltpu.VMEM((CH, D), jnp.float32),
        pltpu.VMEM((CH,), jnp.float32),
        pltpu.VMEM((CH,), jnp.float32),
        pltpu.VMEM((CH,), jnp.int32),
        pltpu.VMEM((CH,), jnp.int32),
        pltpu.VMEM_SHARED((NPAD, D), jnp.float32),
        pltpu.VMEM_SHARED((NPAD,), jnp.float32),
        pltpu.SemaphoreType.DMA,
        pltpu.SemaphoreType.DMA,
        pltpu.SemaphoreType.DMA,
        pltpu.SemaphoreType.DMA,
    ],
)
def _sc_scatter(v_hbm, e_hbm, src_hbm, dst_hbm,
                nacc_out, s_out,
                srcall, vb0, vb1, eb0, eb1, db0, db1, nacc_sh, s_sh,
                sem_g0, sem_g1, sem_s0, sem_s1):
    cid = lax.axis_index("c")
    sid = lax.axis_index("s")
    wid = sid * 2 + cid
    base = wid * IPT
    r0 = sid * ROWS_PT

    pltpu.sync_copy(src_hbm.at[pl.ds(base, IPT)], srcall.at[pl.ds(0, IPT)])

    @pl.when(wid < XTILES)
    def _():
        xoff = EMAIN + wid * CH
        pltpu.sync_copy(src_hbm.at[pl.ds(xoff, CH)], srcall.at[pl.ds(IPT, CH)])

    bufs = ((vb0, eb0, db0, sem_g0, sem_s0),
            (vb1, eb1, db1, sem_g1, sem_s1))

    zv = jnp.zeros((16,), jnp.float32)

    def _zero_row(i, _):
        for c8 in range(D // 16):
            vb0[i, pl.ds(c8 * 16, 16)] = zv
        return 0

    lax.fori_loop(0, CH, _zero_row, 0)
    for g in range(CH // 16):
        eb0[pl.ds(g * 16, 16)] = zv
    for m in range(ROWS_PT // CH):
        pltpu.sync_copy(vb0, nacc_sh.at[pl.ds(r0 + m * CH, CH)])
        pltpu.sync_copy(eb0, s_sh.at[pl.ds(r0 + m * CH, CH)])
    plsc.subcore_barrier()

    def gathers(lj, off, b):
        vb, eb, db, sem_g, _ = bufs[b]
        ssl = srcall.at[pl.ds(lj * CH, CH)]
        return (pltpu.make_async_copy(v_hbm.at[ssl], vb, sem_g),
                pltpu.make_async_copy(e_hbm.at[pl.ds(off, CH)], eb, sem_g),
                pltpu.make_async_copy(dst_hbm.at[pl.ds(off, CH)], db, sem_g))

    def scatters(b):
        vb, eb, db, _, sem_s = bufs[b]
        return (pltpu.make_async_copy(vb, nacc_sh.at[db], sem_s),
                pltpu.make_async_copy(eb, s_sh.at[db], sem_s))

    def start(cps):
        for cp in cps:
            cp.start()

    def wait(cps):
        for cp in cps:
            cp.wait()

    def scale(b):
        vb, eb, _, _, _ = bufs[b]

        def sbody(g, _):
            ev = eb[pl.ds(g * 16, 16)]
            for l in range(16):
                a = ev[l]
                i = g * 16 + l
                for c8 in range(D // 16):
                    sl = pl.ds(c8 * 16, 16)
                    vb[i, sl] = vb[i, sl] * a
            return 0

        lax.fori_loop(0, CH // 16, sbody, 0)

    def retire(j, b):
        wait(gathers(j, base + j * CH, b))
        scale(b)
        vb, eb, db, _, sem_s = bufs[b]
        pltpu.async_copy(vb, nacc_sh.at[db], sem_s, add=True)
        pltpu.async_copy(eb, s_sh.at[db], sem_s, add=True)

    def body(jj, _):
        for b in (0, 1):
            j = 2 * jj + b
            first = (jj == 0) if b == 0 else None

            @pl.when(jj > 0)
            def _():
                wait(scatters(b))

            start(gathers(j, base + j * CH, b))

            if first is None:
                retire(j - 1, 1 - b)
            else:
                pl.when(jj > 0)(lambda: retire(j - 1, 1 - b))
        return 0

    lax.fori_loop(0, CPT // 2, body, 0)

    last = CPT - 1
    retire(last, last % 2)
    wait(scatters((last - 1) % 2))
    wait(scatters(last % 2))

    @pl.when(wid < XTILES)
    def _():
        xoff = EMAIN + wid * CH
        g = gathers(CPT, xoff, 0)
        start(g)
        wait(g)
        scale(0)
        vb, eb, db, _, _ = bufs[0]
        pltpu.sync_copy(vb, nacc_sh.at[db], add=True)
        pltpu.sync_copy(eb, s_sh.at[db], add=True)

    plsc.subcore_barrier()
    pltpu.sync_copy(nacc_sh.at[pl.ds(r0, ROWS_PT)],
                    nacc_out.at[cid, pl.ds(r0, ROWS_PT)])
    pltpu.sync_copy(s_sh.at[pl.ds(r0, ROWS_PT)],
                    s_out.at[cid, pl.ds(r0, ROWS_PT)])


def _layer_sc(prep_outs, src, dst):
    q, k, v, xs, qn2, k2 = prep_outs
    qn2f = qn2.reshape(NPAD)
    k2v = jnp.broadcast_to(k2.reshape(()), (16,))
    e = _sc_score(q, k, qn2f, k2v, src, dst)
    nacc, sacc = _sc_scatter(v, e, src, dst)
    s_col = sacc.reshape(2, NPAD, 1)
    return nacc, s_col[0], s_col[1], xs


def kernel(x, edge_index,
           Wq1, bq1, Wk1, bk1, Wv1, bv1, Ws1, bs1,
           Wq2, bq2, Wk2, bk2, Wv2, bv2, Ws2, bs2,
           Wq3, bq3, Wk3, bk3, Wv3, bv3, Ws3, bs3):
    src = edge_index[0]
    dst = edge_index[1]
    xp = jnp.pad(x, ((0, NPAD - N), (0, 0)))

    def r1(b):
        return b.reshape(1, D)

    p1 = _prep1(xp, Wq1, r1(bq1), Wk1, r1(bk1), Wv1, r1(bv1), Ws1, r1(bs1))
    n, s0, s1, xs1 = _layer_sc(p1, src, dst)

    p2 = _prep23(n, s0, s1, xs1, r1(bs1),
                 Wq2, r1(bq2), Wk2, r1(bk2), Wv2, r1(bv2), Ws2, r1(bs2))
    n, s0, s1, xs2 = _layer_sc(p2, src, dst)

    p3 = _prep23(n, s0, s1, xs2, r1(bs2),
                 Wq3, r1(bq3), Wk3, r1(bk3), Wv3, r1(bv3), Ws3, r1(bs3))
    n, s0, s1, xs3 = _layer_sc(p3, src, dst)

    out = _final(n, s0, s1, xs3, r1(bs3))
    return out[:N]
